# baseline (speedup 1.0000x reference)
import jax
import jax.numpy as jnp
from jax.experimental import pallas as pl
from jax.experimental.pallas import tpu as pltpu

BM = 1024
BN = 2048
BK = 2048


def _mm_kernel(x_ref, w_ref, b_ref, o_ref):
    k = pl.program_id(2)
    acc = jax.lax.dot_general(
        x_ref[...],
        w_ref[...],
        dimension_numbers=(((1,), (1,)), ((), ())),
        preferred_element_type=jnp.float32,
    )

    @pl.when(k == 0)
    def _init():
        o_ref[...] = acc + b_ref[...]

    @pl.when(k != 0)
    def _accum():
        o_ref[...] += acc


def kernel(x, W, b):
    M, K = x.shape
    N = W.shape[0]
    xb = x.astype(jnp.bfloat16)
    Wb = W.astype(jnp.bfloat16)
    b2 = b.reshape(1, N)
    grid = (M // BM, N // BN, K // BK)
    return pl.pallas_call(
        _mm_kernel,
        grid=grid,
        in_specs=[
            pl.BlockSpec((BM, BK), lambda i, j, k: (i, k)),
            pl.BlockSpec((BN, BK), lambda i, j, k: (j, k)),
            pl.BlockSpec((1, BN), lambda i, j, k: (0, j)),
        ],
        out_specs=pl.BlockSpec((BM, BN), lambda i, j, k: (i, j)),
        out_shape=jax.ShapeDtypeStruct((M, N), jnp.float32),
        compiler_params=pltpu.CompilerParams(
            dimension_semantics=("parallel", "parallel", "arbitrary"),
        ),
    )(xb, Wb, b2)


# speedup vs baseline: 1.1531x; 1.1531x over previous
import jax
import jax.numpy as jnp
from jax.experimental import pallas as pl
from jax.experimental.pallas import tpu as pltpu

BM = 2048
BN = 1024
BK = 1024


def _mm_kernel(x_ref, w_ref, b_ref, o_ref):
    k = pl.program_id(2)
    acc = jax.lax.dot_general(
        x_ref[...],
        w_ref[...],
        dimension_numbers=(((1,), (1,)), ((), ())),
        preferred_element_type=jnp.float32,
    )

    @pl.when(k == 0)
    def _init():
        o_ref[...] = acc + b_ref[...]

    @pl.when(k != 0)
    def _accum():
        o_ref[...] += acc


def kernel(x, W, b):
    M, K = x.shape
    N = W.shape[0]
    b2 = b.reshape(1, N)
    grid = (M // BM, N // BN, K // BK)
    return pl.pallas_call(
        _mm_kernel,
        grid=grid,
        in_specs=[
            pl.BlockSpec((BM, BK), lambda i, j, k: (i, k)),
            pl.BlockSpec((BN, BK), lambda i, j, k: (j, k)),
            pl.BlockSpec((1, BN), lambda i, j, k: (0, j)),
        ],
        out_specs=pl.BlockSpec((BM, BN), lambda i, j, k: (i, j)),
        out_shape=jax.ShapeDtypeStruct((M, N), jnp.float32),
        compiler_params=pltpu.CompilerParams(
            dimension_semantics=("parallel", "parallel", "arbitrary"),
        ),
    )(x, W, b2)


# speedup vs baseline: 1.1562x; 1.0028x over previous
import jax
import jax.numpy as jnp
from jax.experimental import pallas as pl
from jax.experimental.pallas import tpu as pltpu

BM = 1024
BN = 2048
BK = 1024


def _mm_kernel(x_ref, w_ref, b_ref, o_ref):
    k = pl.program_id(2)
    acc = jax.lax.dot_general(
        x_ref[...],
        w_ref[...],
        dimension_numbers=(((1,), (1,)), ((), ())),
        preferred_element_type=jnp.float32,
    )

    @pl.when(k == 0)
    def _init():
        o_ref[...] = acc + b_ref[...]

    @pl.when(k != 0)
    def _accum():
        o_ref[...] += acc


def kernel(x, W, b):
    M, K = x.shape
    N = W.shape[0]
    b2 = b.reshape(1, N)
    nj = N // BN
    nk = K // BK
    grid = (M // BM, nj, nk)

    def _snake(i, j, k):
        j_eff = jnp.where(i % 2 == 1, nj - 1 - j, j)
        k_eff = jnp.where(j % 2 == 1, nk - 1 - k, k)
        return j_eff, k_eff

    def _x_map(i, j, k):
        _, k_eff = _snake(i, j, k)
        return i, k_eff

    def _w_map(i, j, k):
        j_eff, k_eff = _snake(i, j, k)
        return j_eff, k_eff

    def _b_map(i, j, k):
        j_eff, _ = _snake(i, j, k)
        return 0, j_eff

    def _o_map(i, j, k):
        j_eff, _ = _snake(i, j, k)
        return i, j_eff

    return pl.pallas_call(
        _mm_kernel,
        grid=grid,
        in_specs=[
            pl.BlockSpec((BM, BK), _x_map),
            pl.BlockSpec((BN, BK), _w_map),
            pl.BlockSpec((1, BN), _b_map),
        ],
        out_specs=pl.BlockSpec((BM, BN), _o_map),
        out_shape=jax.ShapeDtypeStruct((M, N), jnp.float32),
        compiler_params=pltpu.CompilerParams(
            dimension_semantics=("parallel", "parallel", "arbitrary"),
        ),
    )(x, W, b2)
